# baseline (speedup 1.0000x reference)
import functools

import jax
import jax.numpy as jnp
from jax import lax
from jax.experimental import pallas as pl
from jax.experimental.pallas import tpu as pltpu
from jax.experimental.pallas import tpu_sc as plsc

N = 50000
E = 100000
M = 1000
DEG = 4

NC = 2
NS = 16
NW = NC * NS

B = 128
BLOCKS = 13
NP = NW * BLOCKS * B
DP = 128

BN = 256
MP = 1024


def _gather_sum_sc(table, idx2):
    mesh = plsc.VectorSubcoreMesh(core_axis_name="c", subcore_axis_name="s")

    @functools.partial(
        pl.kernel,
        mesh=mesh,
        out_type=jax.ShapeDtypeStruct((NP, DP), jnp.float32),
        scratch_types=[
            pltpu.VMEM((8, 128), jnp.int32),
            pltpu.VMEM((B, DP), jnp.float32),
            pltpu.SemaphoreType.DMA,
        ],
    )
    def body(table_hbm, idx_hbm, out_hbm, idx_v, o_v, sem):
        wid = lax.axis_index("s") * NC + lax.axis_index("c")

        def block(b, carry):
            blk = wid * BLOCKS + b
            base = blk * B
            pltpu.sync_copy(idx_hbm.at[blk], idx_v)
            pltpu.async_copy(table_hbm.at[idx_v.at[0]], o_v, sem).wait()
            descs = [
                pltpu.async_copy(table_hbm.at[idx_v.at[k]], o_v, sem,
                                 add=True)
                for k in range(1, DEG)
            ]
            for d in descs:
                d.wait()
            pltpu.sync_copy(o_v, out_hbm.at[pl.ds(base, B)])
            return carry

        lax.fori_loop(0, BLOCKS, block, 0)

    return body(table, idx2)


def _dense_body(x_ref, a_ref, bs_ref, w1_ref, w2_ref, w3_ref, bias_ref):
    acc = jnp.dot(x_ref[...], w1_ref[...], preferred_element_type=jnp.float32)
    acc = acc + jnp.dot(a_ref[...], w2_ref[...],
                        preferred_element_type=jnp.float32)
    acc = acc + jnp.dot(bs_ref[...], w3_ref[...],
                        preferred_element_type=jnp.float32)
    acc = acc + bias_ref[0:1, :]
    s = jnp.sum(acc * acc, axis=1, keepdims=True)
    nrm = jnp.maximum(jnp.sqrt(s), 1e-12)
    return jnp.maximum(acc / nrm, 0.0)


def _dense_tc(x, asum, bsum, w1, w2, w3, bias):
    np_, dpi = x.shape
    dout = w1.shape[1]
    grid = (np_ // BN,)

    def body(x_ref, a_ref, bs_ref, w1_ref, w2_ref, w3_ref, bias_ref, o_ref):
        o_ref[...] = _dense_body(x_ref, a_ref, bs_ref, w1_ref, w2_ref, w3_ref,
                                 bias_ref)

    return pl.pallas_call(
        body,
        grid=grid,
        in_specs=[
            pl.BlockSpec((BN, dpi), lambda i: (i, 0)),
            pl.BlockSpec((BN, dpi), lambda i: (i, 0)),
            pl.BlockSpec((BN, DP), lambda i: (i, 0)),
            pl.BlockSpec((dpi, dout), lambda i: (0, 0)),
            pl.BlockSpec((dpi, dout), lambda i: (0, 0)),
            pl.BlockSpec((DP, dout), lambda i: (0, 0)),
            pl.BlockSpec((8, dout), lambda i: (0, 0)),
        ],
        out_specs=pl.BlockSpec((BN, dout), lambda i: (i, 0)),
        out_shape=jax.ShapeDtypeStruct((np_, dout), jnp.float32),
    )(x, asum, bsum, w1, w2, w3, bias)


def _dense_seg_tc(x, asum, bsum, w1, w2, w3, bias, ids3):
    np_, dpi = x.shape
    dout = w1.shape[1]
    grid = (np_ // BN,)

    def body(x_ref, a_ref, bs_ref, w1_ref, w2_ref, w3_ref, bias_ref, ids_ref,
             o_ref):
        y = _dense_body(x_ref, a_ref, bs_ref, w1_ref, w2_ref, w3_ref, bias_ref)
        ids = ids_ref[0, 0, :]
        rows = lax.broadcasted_iota(jnp.int32, (MP, BN), 0)
        oh = (rows == ids[None, :]).astype(jnp.bfloat16)
        contrib = jnp.dot(oh, y.astype(jnp.bfloat16),
                          preferred_element_type=jnp.float32)

        @pl.when(pl.program_id(0) == 0)
        def _():
            o_ref[...] = jnp.zeros((MP, dout), jnp.float32)

        o_ref[...] += contrib

    return pl.pallas_call(
        body,
        grid=grid,
        in_specs=[
            pl.BlockSpec((BN, dpi), lambda i: (i, 0)),
            pl.BlockSpec((BN, dpi), lambda i: (i, 0)),
            pl.BlockSpec((BN, DP), lambda i: (i, 0)),
            pl.BlockSpec((dpi, dout), lambda i: (0, 0)),
            pl.BlockSpec((dpi, dout), lambda i: (0, 0)),
            pl.BlockSpec((DP, dout), lambda i: (0, 0)),
            pl.BlockSpec((8, dout), lambda i: (0, 0)),
            pl.BlockSpec((1, 1, BN), lambda i: (i, 0, 0)),
        ],
        out_specs=pl.BlockSpec((MP, dout), lambda i: (0, 0)),
        out_shape=jax.ShapeDtypeStruct((MP, dout), jnp.float32),
    )(x, asum, bsum, w1, w2, w3, bias, ids3)


def _pad2(a, r, c):
    return jnp.pad(a, ((0, r - a.shape[0]), (0, c - a.shape[1])))


def _prep_layer(ws, bs, wd, bd, dpi, dout):
    din = ws.shape[0]
    w1 = _pad2(ws, dpi, dout)
    w2 = _pad2(wd[:din], dpi, dout)
    w3 = _pad2(wd[din:], DP, dout)
    bias = jnp.tile(jnp.pad(bs + bd, (0, dout - bs.shape[0]))[None, :], (8, 1))
    return w1, w2, w3, bias


def kernel(atom_features, bond_features, atom_neighbors, bond_neighbors,
           mol_ids, W_self_0, b_self_0, W_deg_0, b_deg_0, W_self_1, b_self_1,
           W_deg_1, b_deg_1, W_self_2, b_self_2, W_deg_2, b_deg_2):
    d0p, d1p, d3 = DP, DP, 512

    x0 = _pad2(atom_features, NP, d0p)
    bond_t = _pad2(bond_features, E, DP)

    def _idx3(nbr):
        byk = jnp.pad(nbr.astype(jnp.int32),
                      ((0, NP - N), (0, 0))).reshape(NW * BLOCKS, B, DEG)
        byk = jnp.transpose(byk, (0, 2, 1))
        return jnp.pad(byk, ((0, 0), (0, 8 - DEG), (0, 0)))

    anbr = _idx3(atom_neighbors)
    bnbr = _idx3(bond_neighbors)
    ids3 = jnp.pad(mol_ids.astype(jnp.int32), (0, NP - N),
                   constant_values=M).reshape(NP // BN, 1, BN)

    w1_0, w2_0, w3_0, bias0 = _prep_layer(W_self_0, b_self_0, W_deg_0,
                                          b_deg_0, d0p, d1p)
    w1_1, w2_1, w3_1, bias1 = _prep_layer(W_self_1, b_self_1, W_deg_1,
                                          b_deg_1, d1p, d1p)
    w1_2, w2_2, w3_2, bias2 = _prep_layer(W_self_2, b_self_2, W_deg_2,
                                          b_deg_2, d1p, d3)

    bsum = _gather_sum_sc(bond_t, bnbr)
    asum0 = _gather_sum_sc(x0, anbr)
    x1 = _dense_tc(x0, asum0, bsum, w1_0, w2_0, w3_0, bias0)
    asum1 = _gather_sum_sc(x1, anbr)
    x2 = _dense_tc(x1, asum1, bsum, w1_1, w2_1, w3_1, bias1)
    asum2 = _gather_sum_sc(x2, anbr)
    out = _dense_seg_tc(x2, asum2, bsum, w1_2, w2_2, w3_2, bias2, ids3)
    return out[:M]


# speedup vs baseline: 1.5732x; 1.5732x over previous
import functools

import jax
import jax.numpy as jnp
from jax import lax
from jax.experimental import pallas as pl
from jax.experimental.pallas import tpu as pltpu
from jax.experimental.pallas import tpu_sc as plsc

N = 50000
E = 100000
M = 1000
DEG = 4

NC = 2
NS = 16
NW = NC * NS

B = 64
BLOCKS = 25
NP = NW * BLOCKS * B
IDX_CHUNKS = (4 * B) // 128
DP = 128

BN = 256
MP = 1024

BF = jnp.bfloat16


def _gather_sum_sc(table, idx3):
    mesh = plsc.VectorSubcoreMesh(core_axis_name="c", subcore_axis_name="s")

    @functools.partial(
        pl.kernel,
        mesh=mesh,
        out_type=jax.ShapeDtypeStruct((NP, DP), jnp.float32),
        scratch_types=[
            pltpu.VMEM((BLOCKS, IDX_CHUNKS, 128), jnp.int32),
            pltpu.VMEM((4 * B, DP), jnp.float32),
            pltpu.VMEM((4 * B, DP), jnp.float32),
            pltpu.VMEM((B, DP), jnp.float32),
            pltpu.VMEM((B, DP), jnp.float32),
            pltpu.SemaphoreType.DMA,
            pltpu.SemaphoreType.DMA,
            pltpu.SemaphoreType.DMA,
        ],
    )
    def body(table_hbm, idx_hbm, out_hbm, idx_all, g0, g1, o0, o1, sg0, sg1,
             so):
        wid = lax.axis_index("s") * NC + lax.axis_index("c")
        pltpu.sync_copy(idx_hbm.at[pl.ds(wid * BLOCKS, BLOCKS)], idx_all)
        g = (g0, g1)
        o = (o0, o1)
        sg = (sg0, sg1)

        def fire(b, slot):
            return [
                pltpu.async_copy(table_hbm.at[idx_all.at[b, j]],
                                 g[slot].at[pl.ds(j * 128, 128)], sg[slot])
                for j in range(IDX_CHUNKS)
            ]

        gdescs = {0: fire(0, 0)}
        odescs = {}
        for b in range(BLOCKS):
            slot = b & 1
            if b + 1 < BLOCKS:
                gdescs[b + 1] = fire(b + 1, 1 - slot)
            for d in gdescs[b]:
                d.wait()
            if b >= 2:
                odescs[b - 2].wait()
            gv, ov = g[slot], o[slot]

            def rowgrp(i, carry):
                r0 = i * 2
                for rr in range(2):
                    r = r0 + rr
                    for cc in range(DP // 16):
                        sl = pl.ds(cc * 16, 16)
                        ov[r, sl] = ((gv[4 * r, sl] + gv[4 * r + 1, sl])
                                     + (gv[4 * r + 2, sl] + gv[4 * r + 3, sl]))
                return carry

            lax.fori_loop(0, B // 2, rowgrp, 0)
            odescs[b] = pltpu.async_copy(
                ov, out_hbm.at[pl.ds((wid * BLOCKS + b) * B, B)], so)
        odescs[BLOCKS - 2].wait()
        odescs[BLOCKS - 1].wait()

    return body(table, idx3)


def _dense_body(x_ref, a_ref, bs_ref, w1_ref, w2_ref, w3_ref, bias_ref):
    acc = jnp.dot(x_ref[...], w1_ref[...], preferred_element_type=jnp.float32)
    acc = acc + jnp.dot(a_ref[...], w2_ref[...],
                        preferred_element_type=jnp.float32)
    acc = acc + jnp.dot(bs_ref[...], w3_ref[...],
                        preferred_element_type=jnp.float32)
    acc = acc + bias_ref[0:1, :]
    s = jnp.sum(acc * acc, axis=1, keepdims=True)
    nrm = jnp.maximum(jnp.sqrt(s), 1e-12)
    return jnp.maximum(acc / nrm, 0.0)


def _dense_tc(x, asum, bsum, w1, w2, w3, bias):
    np_, dpi = x.shape
    dout = w1.shape[1]
    grid = (np_ // BN,)

    def body(x_ref, a_ref, bs_ref, w1_ref, w2_ref, w3_ref, bias_ref, o_ref):
        o_ref[...] = _dense_body(x_ref, a_ref, bs_ref, w1_ref, w2_ref,
                                 w3_ref, bias_ref)

    return pl.pallas_call(
        body,
        grid=grid,
        in_specs=[
            pl.BlockSpec((BN, dpi), lambda i: (i, 0)),
            pl.BlockSpec((BN, DP), lambda i: (i, 0)),
            pl.BlockSpec((BN, DP), lambda i: (i, 0)),
            pl.BlockSpec((dpi, dout), lambda i: (0, 0)),
            pl.BlockSpec((DP, dout), lambda i: (0, 0)),
            pl.BlockSpec((DP, dout), lambda i: (0, 0)),
            pl.BlockSpec((8, dout), lambda i: (0, 0)),
        ],
        out_specs=pl.BlockSpec((BN, dout), lambda i: (i, 0)),
        out_shape=jax.ShapeDtypeStruct((np_, dout), jnp.float32),
    )(x, asum, bsum, w1, w2, w3, bias)


def _dense_seg_tc(x, asum, bsum, w1, w2, w3, bias, ids3):
    np_, dpi = x.shape
    dout = w1.shape[1]
    grid = (np_ // BN,)

    def body(x_ref, a_ref, bs_ref, w1_ref, w2_ref, w3_ref, bias_ref, ids_ref,
             o_ref):
        y = _dense_body(x_ref, a_ref, bs_ref, w1_ref, w2_ref, w3_ref,
                        bias_ref)
        ids = ids_ref[0, 0, :]
        rows = lax.broadcasted_iota(jnp.int32, (MP, BN), 0)
        oh = (rows == ids[None, :]).astype(BF)
        contrib = jnp.dot(oh, y.astype(BF),
                          preferred_element_type=jnp.float32)

        @pl.when(pl.program_id(0) == 0)
        def _():
            o_ref[...] = jnp.zeros((MP, dout), jnp.float32)

        o_ref[...] += contrib

    return pl.pallas_call(
        body,
        grid=grid,
        in_specs=[
            pl.BlockSpec((BN, dpi), lambda i: (i, 0)),
            pl.BlockSpec((BN, DP), lambda i: (i, 0)),
            pl.BlockSpec((BN, DP), lambda i: (i, 0)),
            pl.BlockSpec((dpi, dout), lambda i: (0, 0)),
            pl.BlockSpec((DP, dout), lambda i: (0, 0)),
            pl.BlockSpec((DP, dout), lambda i: (0, 0)),
            pl.BlockSpec((8, dout), lambda i: (0, 0)),
            pl.BlockSpec((1, 1, BN), lambda i: (i, 0, 0)),
        ],
        out_specs=pl.BlockSpec((MP, dout), lambda i: (0, 0)),
        out_shape=jax.ShapeDtypeStruct((MP, dout), jnp.float32),
    )(x, asum, bsum, w1, w2, w3, bias, ids3)


def _pad2(a, r, c):
    return jnp.pad(a, ((0, r - a.shape[0]), (0, c - a.shape[1])))


def _prep_layer(ws, bs, wd, bd, dpi, dout):
    din = ws.shape[0]
    w1 = _pad2(ws, dpi, dout)
    w2 = _pad2(wd[:din], DP, dout)
    w3 = _pad2(wd[din:], DP, dout)
    bias = jnp.tile(jnp.pad(bs + bd, (0, dout - bs.shape[0]))[None, :], (8, 1))
    return w1, w2, w3, bias


def kernel(atom_features, bond_features, atom_neighbors, bond_neighbors,
           mol_ids, W_self_0, b_self_0, W_deg_0, b_deg_0, W_self_1, b_self_1,
           W_deg_1, b_deg_1, W_self_2, b_self_2, W_deg_2, b_deg_2):
    d3 = 512

    x0 = _pad2(atom_features, NP, DP)
    bond_t = _pad2(bond_features, E, DP)

    def _idx3(nbr):
        flat = jnp.pad(nbr.astype(jnp.int32),
                       ((0, NP - N), (0, 0))).reshape(NW * BLOCKS, 4 * B)
        return flat.reshape(NW * BLOCKS, IDX_CHUNKS, 128)

    anbr = _idx3(atom_neighbors)
    bnbr = _idx3(bond_neighbors)
    ids3 = jnp.pad(mol_ids.astype(jnp.int32), (0, NP - N),
                   constant_values=M).reshape(NP // BN, 1, BN)

    w1_0, w2_0, w3_0, bias0 = _prep_layer(W_self_0, b_self_0, W_deg_0,
                                          b_deg_0, DP, DP)
    w1_1, w2_1, w3_1, bias1 = _prep_layer(W_self_1, b_self_1, W_deg_1,
                                          b_deg_1, DP, DP)
    w1_2, w2_2, w3_2, bias2 = _prep_layer(W_self_2, b_self_2, W_deg_2,
                                          b_deg_2, DP, d3)

    bsum = _gather_sum_sc(bond_t, bnbr)
    asum0 = _gather_sum_sc(x0, anbr)
    x1 = _dense_tc(x0, asum0, bsum, w1_0, w2_0, w3_0, bias0)
    asum1 = _gather_sum_sc(x1, anbr)
    x2 = _dense_tc(x1, asum1, bsum, w1_1, w2_1, w3_1, bias1)
    asum2 = _gather_sum_sc(x2, anbr)
    out = _dense_seg_tc(x2, asum2, bsum, w1_2, w2_2, w3_2, bias2, ids3)
    return out[:M]


# speedup vs baseline: 1.5835x; 1.0065x over previous
import functools

import jax
import jax.numpy as jnp
from jax import lax
from jax.experimental import pallas as pl
from jax.experimental.pallas import tpu as pltpu
from jax.experimental.pallas import tpu_sc as plsc

N = 50000
E = 100000
M = 1000
DEG = 4

NC = 2
NS = 16
NW = NC * NS

B = 64
BLOCKS = 25
NP = NW * BLOCKS * B
IDX_CHUNKS = (4 * B) // 128
RING = 3
DP = 128

BN = 256
MP = 1024

BF = jnp.bfloat16


def _gather_sum_sc(table, idx3):
    mesh = plsc.VectorSubcoreMesh(core_axis_name="c", subcore_axis_name="s")

    @functools.partial(
        pl.kernel,
        mesh=mesh,
        out_type=jax.ShapeDtypeStruct((NP, DP), jnp.float32),
        scratch_types=(
            [pltpu.VMEM((BLOCKS, IDX_CHUNKS, 128), jnp.int32)]
            + [pltpu.VMEM((4 * B, DP), jnp.float32) for _ in range(RING)]
            + [pltpu.SemaphoreType.DMA for _ in range(RING)]
            + [pltpu.SemaphoreType.DMA]
        ),
    )
    def body(table_hbm, idx_hbm, out_hbm, idx_all, *rest):
        g = rest[:RING]
        sg = rest[RING:2 * RING]
        so = rest[2 * RING]
        wid = lax.axis_index("s") * NC + lax.axis_index("c")
        pltpu.sync_copy(idx_hbm.at[pl.ds(wid * BLOCKS, BLOCKS)], idx_all)

        def fire(b):
            slot = b % RING
            return [
                pltpu.async_copy(table_hbm.at[idx_all.at[b, j]],
                                 g[slot].at[pl.ds(j * 128, 128)], sg[slot])
                for j in range(IDX_CHUNKS)
            ]

        gdescs = {b: fire(b) for b in range(RING - 1)}
        odescs = {}
        for b in range(BLOCKS):
            slot = b % RING
            if b + RING - 1 < BLOCKS:
                if b >= 1:
                    odescs[b - 1].wait()
                gdescs[b + RING - 1] = fire(b + RING - 1)
            for d in gdescs[b]:
                d.wait()
            gv = g[slot]

            def rowsum(r, carry):
                for cc in range(DP // 16):
                    sl = pl.ds(cc * 16, 16)
                    gv[r, sl] = ((gv[4 * r, sl] + gv[4 * r + 1, sl])
                                 + (gv[4 * r + 2, sl] + gv[4 * r + 3, sl]))
                return carry

            lax.fori_loop(0, B, rowsum, 0, unroll=2)
            odescs[b] = pltpu.async_copy(
                gv.at[pl.ds(0, B)],
                out_hbm.at[pl.ds((wid * BLOCKS + b) * B, B)], so)
        for b in range(BLOCKS - RING, BLOCKS):
            odescs[b].wait()

    return body(table, idx3)


def _dense_body(x_ref, a_ref, bs_ref, w1_ref, w2_ref, w3_ref, bias_ref):
    acc = jnp.dot(x_ref[...], w1_ref[...], preferred_element_type=jnp.float32)
    acc = acc + jnp.dot(a_ref[...], w2_ref[...],
                        preferred_element_type=jnp.float32)
    acc = acc + jnp.dot(bs_ref[...], w3_ref[...],
                        preferred_element_type=jnp.float32)
    acc = acc + bias_ref[0:1, :]
    s = jnp.sum(acc * acc, axis=1, keepdims=True)
    nrm = jnp.maximum(jnp.sqrt(s), 1e-12)
    return jnp.maximum(acc / nrm, 0.0)


def _dense_tc(x, asum, bsum, w1, w2, w3, bias):
    np_, dpi = x.shape
    dout = w1.shape[1]
    grid = (np_ // BN,)

    def body(x_ref, a_ref, bs_ref, w1_ref, w2_ref, w3_ref, bias_ref, o_ref):
        o_ref[...] = _dense_body(x_ref, a_ref, bs_ref, w1_ref, w2_ref,
                                 w3_ref, bias_ref)

    return pl.pallas_call(
        body,
        grid=grid,
        in_specs=[
            pl.BlockSpec((BN, dpi), lambda i: (i, 0)),
            pl.BlockSpec((BN, DP), lambda i: (i, 0)),
            pl.BlockSpec((BN, DP), lambda i: (i, 0)),
            pl.BlockSpec((dpi, dout), lambda i: (0, 0)),
            pl.BlockSpec((DP, dout), lambda i: (0, 0)),
            pl.BlockSpec((DP, dout), lambda i: (0, 0)),
            pl.BlockSpec((8, dout), lambda i: (0, 0)),
        ],
        out_specs=pl.BlockSpec((BN, dout), lambda i: (i, 0)),
        out_shape=jax.ShapeDtypeStruct((np_, dout), jnp.float32),
    )(x, asum, bsum, w1, w2, w3, bias)


def _dense_seg_tc(x, asum, bsum, w1, w2, w3, bias, ids3):
    np_, dpi = x.shape
    dout = w1.shape[1]
    grid = (np_ // BN,)

    def body(x_ref, a_ref, bs_ref, w1_ref, w2_ref, w3_ref, bias_ref, ids_ref,
             o_ref):
        y = _dense_body(x_ref, a_ref, bs_ref, w1_ref, w2_ref, w3_ref,
                        bias_ref)
        ids = ids_ref[0, 0, :]
        rows = lax.broadcasted_iota(jnp.int32, (MP, BN), 0)
        oh = (rows == ids[None, :]).astype(BF)
        contrib = jnp.dot(oh, y.astype(BF),
                          preferred_element_type=jnp.float32)

        @pl.when(pl.program_id(0) == 0)
        def _():
            o_ref[...] = jnp.zeros((MP, dout), jnp.float32)

        o_ref[...] += contrib

    return pl.pallas_call(
        body,
        grid=grid,
        in_specs=[
            pl.BlockSpec((BN, dpi), lambda i: (i, 0)),
            pl.BlockSpec((BN, DP), lambda i: (i, 0)),
            pl.BlockSpec((BN, DP), lambda i: (i, 0)),
            pl.BlockSpec((dpi, dout), lambda i: (0, 0)),
            pl.BlockSpec((DP, dout), lambda i: (0, 0)),
            pl.BlockSpec((DP, dout), lambda i: (0, 0)),
            pl.BlockSpec((8, dout), lambda i: (0, 0)),
            pl.BlockSpec((1, 1, BN), lambda i: (i, 0, 0)),
        ],
        out_specs=pl.BlockSpec((MP, dout), lambda i: (0, 0)),
        out_shape=jax.ShapeDtypeStruct((MP, dout), jnp.float32),
    )(x, asum, bsum, w1, w2, w3, bias, ids3)


def _pad2(a, r, c):
    return jnp.pad(a, ((0, r - a.shape[0]), (0, c - a.shape[1])))


def _prep_layer(ws, bs, wd, bd, dpi, dout):
    din = ws.shape[0]
    w1 = _pad2(ws, dpi, dout)
    w2 = _pad2(wd[:din], DP, dout)
    w3 = _pad2(wd[din:], DP, dout)
    bias = jnp.tile(jnp.pad(bs + bd, (0, dout - bs.shape[0]))[None, :], (8, 1))
    return w1, w2, w3, bias


def kernel(atom_features, bond_features, atom_neighbors, bond_neighbors,
           mol_ids, W_self_0, b_self_0, W_deg_0, b_deg_0, W_self_1, b_self_1,
           W_deg_1, b_deg_1, W_self_2, b_self_2, W_deg_2, b_deg_2):
    d3 = 512

    x0 = _pad2(atom_features, NP, DP)
    bond_t = _pad2(bond_features, E, DP)

    def _idx3(nbr):
        flat = jnp.pad(nbr.astype(jnp.int32),
                       ((0, NP - N), (0, 0))).reshape(NW * BLOCKS, 4 * B)
        return flat.reshape(NW * BLOCKS, IDX_CHUNKS, 128)

    anbr = _idx3(atom_neighbors)
    bnbr = _idx3(bond_neighbors)
    ids3 = jnp.pad(mol_ids.astype(jnp.int32), (0, NP - N),
                   constant_values=M).reshape(NP // BN, 1, BN)

    w1_0, w2_0, w3_0, bias0 = _prep_layer(W_self_0, b_self_0, W_deg_0,
                                          b_deg_0, DP, DP)
    w1_1, w2_1, w3_1, bias1 = _prep_layer(W_self_1, b_self_1, W_deg_1,
                                          b_deg_1, DP, DP)
    w1_2, w2_2, w3_2, bias2 = _prep_layer(W_self_2, b_self_2, W_deg_2,
                                          b_deg_2, DP, d3)

    bsum = _gather_sum_sc(bond_t, bnbr)
    asum0 = _gather_sum_sc(x0, anbr)
    x1 = _dense_tc(x0, asum0, bsum, w1_0, w2_0, w3_0, bias0)
    asum1 = _gather_sum_sc(x1, anbr)
    x2 = _dense_tc(x1, asum1, bsum, w1_1, w2_1, w3_1, bias1)
    asum2 = _gather_sum_sc(x2, anbr)
    out = _dense_seg_tc(x2, asum2, bsum, w1_2, w2_2, w3_2, bias2, ids3)
    return out[:M]
